# baseline (speedup 1.0000x reference)
import functools

import jax
import jax.numpy as jnp
from jax import lax
from jax.experimental import pallas as pl
from jax.experimental.pallas import tpu as pltpu
from jax.experimental.pallas import tpu_sc as plsc

_CHUNK = 40
_TPM = 5
_D = 128
_K = 3 * _D
_NKV = _K // 16
_OPT = 32

_mesh = plsc.VectorSubcoreMesh(core_axis_name="c", subcore_axis_name="s")


def _lane_perm(x, idx):
    return lax.gather(
        x, idx[:, None],
        dimension_numbers=lax.GatherDimensionNumbers(
            offset_dims=(), collapsed_slice_dims=(0,), start_index_map=(0,)),
        slice_sizes=(1,),
        mode=lax.GatherScatterMode.PROMISE_IN_BOUNDS)


@functools.partial(
    pl.kernel,
    mesh=_mesh,
    out_type=(
        jax.ShapeDtypeStruct((1000,), jnp.float32),
        jax.ShapeDtypeStruct((2, 16), jnp.float32),
    ),
    scratch_types=[
        pltpu.VMEM((_CHUNK,), jnp.int32),
        pltpu.VMEM((_CHUNK,), jnp.int32),
        pltpu.VMEM((_CHUNK,), jnp.int32),
        pltpu.VMEM((_CHUNK, _D), jnp.float32),
        pltpu.VMEM((_CHUNK, _D), jnp.float32),
        pltpu.VMEM((_CHUNK, _D), jnp.float32),
        pltpu.VMEM((_K,), jnp.float32),
        pltpu.VMEM((_TPM, _K), jnp.float32),
        pltpu.VMEM((_OPT, _K), jnp.float32),
        pltpu.VMEM((_OPT,), jnp.float32),
        pltpu.VMEM((16, _D), jnp.float32),
        pltpu.VMEM((16,), jnp.float32),
        pltpu.VMEM_SHARED((8, _K), jnp.float32),
        pltpu.VMEM_SHARED((16, _D), jnp.float32),
        pltpu.SemaphoreType.DMA,
        pltpu.SemaphoreType.DMA,
    ],
)
def _fused(pat_hbm, e0, e1, e2, w_hbm, b_hbm, res_hbm, bn_hbm,
           idx0_v, idx1_v, idx2_v, rows0_v, rows1_v, rows2_v,
           acc_v, part_v, w_v, b_v, sigrb, vtmp,
           part_sh, sig_sh, sem_w, sem_g):
    c = lax.axis_index("c")
    s = lax.axis_index("s")
    tail = jnp.logical_and(c == 1, s == 15)
    base = pl.multiple_of(jnp.where(tail, 1000 - _OPT, 512 * c + _OPT * s), 8)

    cw = pltpu.async_copy(w_hbm.at[pl.ds(base, _OPT)], w_v, sem_w)
    cb = pltpu.async_copy(b_hbm.at[pl.ds(base, _OPT)], b_v, sem_w)

    @pl.when(s < _TPM)
    def _():
        off = pl.multiple_of(s * _CHUNK, 8)
        pltpu.sync_copy(pat_hbm.at[pl.ds(600 + off, _CHUNK)], idx0_v)
        pltpu.sync_copy(pat_hbm.at[pl.ds(800 + off, _CHUNK)], idx1_v)
        pltpu.sync_copy(pat_hbm.at[pl.ds(400 + off, _CHUNK)], idx2_v)
        g0 = pltpu.async_copy(e0.at[idx0_v], rows0_v, sem_g)
        g1 = pltpu.async_copy(e1.at[idx1_v], rows1_v, sem_g)
        g2 = pltpu.async_copy(e2.at[idx2_v], rows2_v, sem_g)
        g0.wait()
        g1.wait()
        g2.wait()
        for m, rv in enumerate((rows0_v, rows1_v, rows2_v)):
            for v in range(_D // 16):
                a = rv[0, pl.ds(v * 16, 16)]
                for r in range(1, _CHUNK):
                    a = a + rv[r, pl.ds(v * 16, 16)]
                acc_v[pl.ds(m * _D + v * 16, 16)] = a
        pltpu.sync_copy(acc_v, part_sh.at[s])

    plsc.subcore_barrier()

    pltpu.sync_copy(part_sh.at[pl.ds(0, _TPM)], part_v)
    cw.wait()
    cb.wait()

    rep = []
    for v in range(_NKV):
        a = part_v[0, pl.ds(v * 16, 16)]
        for p in range(1, _TPM):
            a = a + part_v[p, pl.ds(v * 16, 16)]
        rep.append(jnp.maximum(a, 0.0))

    iota = jnp.arange(16, dtype=jnp.int32)
    own_lo = jnp.where(tail, 992, base)
    perm_idx = [jnp.bitwise_xor(iota, 1 << k) for k in range(4)]
    fold_msk = [(jnp.bitwise_and(jnp.right_shift(iota, k), 1)) == 0
                for k in range(4)]
    ssum = jnp.zeros((16,), jnp.float32)
    for ch in range(2):
        vecs = []
        for j in range(16):
            o = ch * 16 + j
            a = rep[0] * w_v[o, pl.ds(0, 16)]
            for v in range(1, _NKV):
                a = a + rep[v] * w_v[o, pl.ds(v * 16, 16)]
            vecs.append(a)
        for k in range(4):
            nxt = []
            for i in range(len(vecs) // 2):
                x, y = vecs[2 * i], vecs[2 * i + 1]
                nxt.append(
                    jnp.where(fold_msk[k], x, _lane_perm(y, perm_idx[k]))
                    + jnp.where(fold_msk[k], _lane_perm(x, perm_idx[k]), y))
            vecs = nxt
        r = vecs[0] + b_v[pl.ds(ch * 16, 16)]
        vtmp[...] = r
        pltpu.sync_copy(vtmp, res_hbm.at[pl.ds(base + ch * 16, 16)])
        glob = base + ch * 16 + iota
        sig = 1.0 / (1.0 + jnp.exp(-r))
        mask = jnp.logical_and(glob >= own_lo, glob < 1000)
        ssum = ssum + jnp.where(mask, sig, 0.0)

    acc_v[pl.ds(0, 16)] = ssum
    zero16 = jnp.zeros((16,), jnp.float32)
    for v in range(1, _D // 16):
        acc_v[pl.ds(v * 16, 16)] = zero16
    pltpu.sync_copy(acc_v.at[pl.ds(0, _D)], sig_sh.at[s])
    plsc.subcore_barrier()

    @pl.when(s == 0)
    def _():
        pltpu.sync_copy(sig_sh, sigrb)
        tot = sigrb[0, pl.ds(0, 16)]
        for j in range(1, 16):
            tot = tot + sigrb[j, pl.ds(0, 16)]
        for k in range(4):
            tot = tot + _lane_perm(tot, perm_idx[k])
        vtmp[...] = tot
        pltpu.sync_copy(vtmp, bn_hbm.at[c])


def kernel(patient, E0, E1, E2, W, b, ddi_adj):
    pat = patient.reshape(-1).astype(jnp.int32)
    res, bnp = _fused(pat, E0, E1, E2, W, b)
    stot = bnp[0, 0] + bnp[1, 0]
    bn = 0.0005 * ddi_adj[0, 0] * stot * stot
    return (res.reshape(1, 1000), bn)


# speedup vs baseline: 1.2455x; 1.2455x over previous
import functools

import jax
import jax.numpy as jnp
from jax import lax
from jax.experimental import pallas as pl
from jax.experimental.pallas import tpu as pltpu
from jax.experimental.pallas import tpu_sc as plsc

_CHUNK = 40
_NJOB = 5
_D = 128
_K = 3 * _D

_mesh = plsc.VectorSubcoreMesh(core_axis_name="c", subcore_axis_name="s")


@functools.partial(
    pl.kernel,
    mesh=_mesh,
    out_type=jax.ShapeDtypeStruct((_NJOB, _K), jnp.float32),
    scratch_types=[
        pltpu.VMEM((_CHUNK,), jnp.int32),
        pltpu.VMEM((_CHUNK,), jnp.int32),
        pltpu.VMEM((_CHUNK,), jnp.int32),
        pltpu.VMEM((_CHUNK, _D), jnp.float32),
        pltpu.VMEM((_CHUNK, _D), jnp.float32),
        pltpu.VMEM((_CHUNK, _D), jnp.float32),
        pltpu.VMEM((_K,), jnp.float32),
        pltpu.SemaphoreType.DMA,
    ],
)
def _gather_pool(pat_hbm, e0, e1, e2, out_hbm,
                 idx0_v, idx1_v, idx2_v, rows0_v, rows1_v, rows2_v,
                 acc_v, sem_g):
    c = lax.axis_index("c")
    s = lax.axis_index("s")
    job = s * 2 + c

    @pl.when(job < _NJOB)
    def _():
        off = pl.multiple_of(job * _CHUNK, 8)
        pltpu.sync_copy(pat_hbm.at[pl.ds(600 + off, _CHUNK)], idx0_v)
        pltpu.sync_copy(pat_hbm.at[pl.ds(800 + off, _CHUNK)], idx1_v)
        pltpu.sync_copy(pat_hbm.at[pl.ds(400 + off, _CHUNK)], idx2_v)
        g0 = pltpu.async_copy(e0.at[idx0_v], rows0_v, sem_g)
        g1 = pltpu.async_copy(e1.at[idx1_v], rows1_v, sem_g)
        g2 = pltpu.async_copy(e2.at[idx2_v], rows2_v, sem_g)
        g0.wait()
        g1.wait()
        g2.wait()
        for m, rv in enumerate((rows0_v, rows1_v, rows2_v)):
            for v in range(_D // 16):
                a = rv[0, pl.ds(v * 16, 16)]
                for r in range(1, _CHUNK):
                    a = a + rv[r, pl.ds(v * 16, 16)]
                acc_v[pl.ds(m * _D + v * 16, 16)] = a
        pltpu.sync_copy(acc_v, out_hbm.at[job])


def _dense(partial_ref, w_ref, b_ref, ddi_ref, res_ref, bn_ref):
    rep = jnp.sum(partial_ref[:], axis=0, keepdims=True)
    rep = jnp.maximum(rep, 0.0)
    out = lax.dot_general(
        rep, w_ref[:],
        dimension_numbers=(((1,), (1,)), ((), ())),
        preferred_element_type=jnp.float32,
    ) + b_ref[:]
    res_ref[:] = out
    neg = jax.nn.sigmoid(out)
    s = jnp.sum(neg)
    bn_ref[:] = jnp.reshape(0.0005 * ddi_ref[0, 0] * s * s, (1, 1))


def kernel(patient, E0, E1, E2, W, b, ddi_adj):
    partial = _gather_pool(patient.reshape(-1), E0, E1, E2)
    result, bn = pl.pallas_call(
        _dense,
        out_shape=(
            jax.ShapeDtypeStruct((1, 1000), jnp.float32),
            jax.ShapeDtypeStruct((1, 1), jnp.float32),
        ),
    )(partial, W, b.reshape(1, 1000), ddi_adj)
    return (result, bn.reshape(()))


# speedup vs baseline: 1.4670x; 1.1778x over previous
import functools

import jax
import jax.numpy as jnp
from jax import lax
from jax.experimental import pallas as pl
from jax.experimental.pallas import tpu as pltpu
from jax.experimental.pallas import tpu_sc as plsc

_CHUNK = 40
_NJOB = 5
_D = 128
_K = 3 * _D

_mesh = plsc.VectorSubcoreMesh(core_axis_name="c", subcore_axis_name="s")


_TPM = 5


@functools.partial(
    pl.kernel,
    mesh=_mesh,
    out_type=jax.ShapeDtypeStruct((_NJOB, _K), jnp.float32),
    scratch_types=[
        pltpu.VMEM((_CHUNK,), jnp.int32),
        pltpu.VMEM((_CHUNK, _D), jnp.float32),
        pltpu.VMEM((_D,), jnp.float32),
        pltpu.SemaphoreType.DMA,
    ],
)
def _gather_pool(pat_hbm, e0, e1, e2, out_hbm, idx_v, rows_v, acc_v, sem_g):
    wid = lax.axis_index("s") * 2 + lax.axis_index("c")

    @pl.when(wid < 3 * _TPM)
    def _():
        m = wid // _TPM
        p = wid % _TPM
        off = 600 + 200 * m - 600 * (m // 2) + _CHUNK * p
        pltpu.sync_copy(pat_hbm.at[pl.ds(pl.multiple_of(off, 8), _CHUNK)],
                        idx_v)

        @pl.when(m == 0)
        def _():
            pltpu.async_copy(e0.at[idx_v], rows_v, sem_g).wait()

        @pl.when(m == 1)
        def _():
            pltpu.async_copy(e1.at[idx_v], rows_v, sem_g).wait()

        @pl.when(m == 2)
        def _():
            pltpu.async_copy(e2.at[idx_v], rows_v, sem_g).wait()

        for v in range(_D // 16):
            a = rows_v[0, pl.ds(v * 16, 16)]
            for r in range(1, _CHUNK):
                a = a + rows_v[r, pl.ds(v * 16, 16)]
            acc_v[pl.ds(v * 16, 16)] = a
        pltpu.sync_copy(acc_v, out_hbm.at[p, pl.ds(pl.multiple_of(m * _D, 8),
                                                   _D)])


def _dense(partial_ref, w_ref, b_ref, ddi_ref, res_ref, bn_ref):
    rep = jnp.sum(partial_ref[:], axis=0, keepdims=True)
    rep = jnp.maximum(rep, 0.0)
    out = lax.dot_general(
        rep, w_ref[:],
        dimension_numbers=(((1,), (1,)), ((), ())),
        preferred_element_type=jnp.float32,
    ) + b_ref[:]
    res_ref[:] = out
    neg = jax.nn.sigmoid(out)
    s = jnp.sum(neg)
    bn_ref[:] = jnp.reshape(0.0005 * ddi_ref[0, 0] * s * s, (1, 1))


def kernel(patient, E0, E1, E2, W, b, ddi_adj):
    partial = _gather_pool(patient.reshape(-1), E0, E1, E2)
    result, bn = pl.pallas_call(
        _dense,
        out_shape=(
            jax.ShapeDtypeStruct((1, 1000), jnp.float32),
            jax.ShapeDtypeStruct((1, 1), jnp.float32),
        ),
    )(partial, W, b.reshape(1, 1000), ddi_adj)
    return (result, bn.reshape(()))
